# baseline (speedup 1.0000x reference)
import functools

import jax
import jax.numpy as jnp
from jax import lax
from jax.experimental import pallas as pl
from jax.experimental.pallas import tpu as pltpu
from jax.experimental.pallas import tpu_sc as plsc

VOCAB_ = 1000000
B_ = 16384
D_ = 64
PW_ = 128
NEG_ = 20
L_ = 16
NC_ = 2
NS_ = 16
NW_ = NC_ * NS_
CB_ = B_ // NW_
G_ = 16
NCHUNK_ = CB_ // G_
PAIRS_ = G_ * NEG_
SEGS_ = (128, 128, 64)
KD_ = D_ // L_

TRB_ = 3968
NTRB_ = 127
SSPLIT_ = TRB_ * NTRB_


def _tr_body(xa_ref, xb_ref, o_ref):
    eye = jnp.eye(D_, dtype=jnp.float32)
    dims = (((0,), (0,)), ((), ()))
    ya = lax.dot_general(xa_ref[...], eye, dims,
                         preferred_element_type=jnp.float32)
    yb = lax.dot_general(xb_ref[...], eye, dims,
                         preferred_element_type=jnp.float32)
    o_ref[...] = jnp.concatenate([ya, yb], axis=1)


_tr = pl.pallas_call(
    _tr_body,
    grid=(NTRB_,),
    in_specs=[
        pl.BlockSpec((D_, TRB_), lambda i: (0, i)),
        pl.BlockSpec((D_, TRB_),
                     lambda i: (0, jnp.minimum(i + NTRB_, VOCAB_ // TRB_))),
    ],
    out_specs=pl.BlockSpec((TRB_, PW_), lambda i: (i, 0)),
    out_shape=jax.ShapeDtypeStruct((SSPLIT_, PW_), jnp.float32),
)


def _to_rows(table):
    tt = jnp.swapaxes(table, 0, 1)
    return _tr(tt, tt)


def _sc_body(w_hbm, c_hbm, negs_hbm, in_hbm, out_hbm,
             pos_hbm, neg_hbm,
             idx_w, idx_c, idx_n, row_w, row_c, row_n,
             wi_v, wo_v, wn_v, pos_buf, neg_buf, sem):
    cid = lax.axis_index("c")
    sid = lax.axis_index("s")
    wid = sid * NC_ + cid
    base = wid * CB_
    nbase = wid * (CB_ * NEG_)

    pltpu.sync_copy(w_hbm.at[pl.ds(base, CB_)], idx_w.at[pl.ds(0, CB_)])
    pltpu.sync_copy(c_hbm.at[pl.ds(base, CB_)], idx_c.at[pl.ds(0, CB_)])
    pltpu.sync_copy(negs_hbm.at[pl.ds(nbase, CB_ * NEG_)],
                    idx_n.at[pl.ds(0, CB_ * NEG_)])

    def mk_rows(i, c2):
        tw = idx_w[pl.ds(i * L_, L_)]
        row_w[pl.ds(i * L_, L_)] = tw - jnp.where(tw >= SSPLIT_, SSPLIT_, 0)
        tc = idx_c[pl.ds(i * L_, L_)]
        row_c[pl.ds(i * L_, L_)] = tc - jnp.where(tc >= SSPLIT_, SSPLIT_, 0)
        return c2
    lax.fori_loop(0, CB_ // L_, mk_rows, 0)

    def mk_rows_n(i, c2):
        tn = idx_n[pl.ds(i * L_, L_)]
        row_n[pl.ds(i * L_, L_)] = tn - jnp.where(tn >= SSPLIT_, SSPLIT_, 0)
        return c2
    lax.fori_loop(0, (CB_ * NEG_) // L_, mk_rows_n, 0)

    iota = lax.iota(jnp.int32, L_)

    def chunk(g, carry):
        cps = [
            pltpu.async_copy(in_hbm.at[row_w.at[pl.ds(g * G_, G_)]], wi_v, sem),
            pltpu.async_copy(out_hbm.at[row_c.at[pl.ds(g * G_, G_)]], wo_v, sem),
        ]
        off = 0
        for seg in SEGS_:
            cps.append(pltpu.async_copy(
                out_hbm.at[row_n.at[pl.ds(g * PAIRS_ + off, seg)]],
                wn_v.at[pl.ds(off, seg)], sem))
            off += seg
        for cp in cps:
            cp.wait()

        def elem(j, carry2):
            acc_neg, acc_pos = carry2
            vw = idx_w[pl.ds(g * G_ + j, L_)]
            hw = jnp.where(vw[0] >= SSPLIT_, D_, 0)
            wis = [wi_v[j, pl.ds(hw + k * L_, L_)] for k in range(KD_)]
            vc = idx_c[pl.ds(g * G_ + j, L_)]
            hc = jnp.where(vc[0] >= SSPLIT_, D_, 0)
            pacc = wo_v[j, pl.ds(hc, L_)] * wis[0]
            for k in range(1, KD_):
                pacc = pacc + wo_v[j, pl.ds(hc + k * L_, L_)] * wis[k]
            lane_p = j & (L_ - 1)
            acc_pos = jnp.where(iota == lane_p, jnp.sum(pacc), acc_pos)
            pos_buf[pl.ds(g * G_ + j - lane_p, L_)] = acc_pos
            p0 = g * PAIRS_ + j * NEG_
            vn0 = idx_n[pl.ds(p0, L_)]
            vn1 = idx_n[pl.ds(p0 + 8, L_)]
            for n in range(NEG_):
                p = j * NEG_ + n
                hbit = vn0[n] if n < L_ else vn1[n - 8]
                hn = jnp.where(hbit >= SSPLIT_, D_, 0)
                a = wn_v[p, pl.ds(hn, L_)] * wis[0]
                for k in range(1, KD_):
                    a = a + wn_v[p, pl.ds(hn + k * L_, L_)] * wis[k]
                lane = p & (L_ - 1)
                acc_neg = jnp.where(iota == lane, jnp.sum(a), acc_neg)
                neg_buf[pl.ds(g * PAIRS_ + p - lane, L_)] = acc_neg
            return (acc_neg, acc_pos)
        zero = jnp.zeros((L_,), jnp.float32)
        lax.fori_loop(0, G_, elem, (zero, zero))
        return carry

    lax.fori_loop(0, NCHUNK_, chunk, 0)

    def sig_pos(i, c2):
        v = pos_buf[pl.ds(i * L_, L_)]
        pos_buf[pl.ds(i * L_, L_)] = 1.0 / (1.0 + jnp.exp(-v))
        return c2
    lax.fori_loop(0, CB_ // L_, sig_pos, 0)

    def sig_neg(i, c2):
        v = neg_buf[pl.ds(i * L_, L_)]
        neg_buf[pl.ds(i * L_, L_)] = 1.0 / (1.0 + jnp.exp(v))
        return c2
    lax.fori_loop(0, (CB_ * NEG_) // L_, sig_neg, 0)

    pltpu.sync_copy(pos_buf, pos_hbm.at[pl.ds(base, CB_)])
    pltpu.sync_copy(neg_buf, neg_hbm.at[pl.ds(nbase, CB_ * NEG_)])


_sc_call = functools.partial(
    pl.kernel,
    out_type=(
        jax.ShapeDtypeStruct((B_,), jnp.float32),
        jax.ShapeDtypeStruct((B_ * NEG_,), jnp.float32),
    ),
    mesh=plsc.VectorSubcoreMesh(core_axis_name="c", subcore_axis_name="s"),
    compiler_params=pltpu.CompilerParams(
        needs_layout_passes=False, use_tc_tiling_on_sc=False),
    scratch_types=[
        pltpu.VMEM((CB_ + L_,), jnp.int32),
        pltpu.VMEM((CB_ + L_,), jnp.int32),
        pltpu.VMEM((CB_ * NEG_ + L_,), jnp.int32),
        pltpu.VMEM((CB_,), jnp.int32),
        pltpu.VMEM((CB_,), jnp.int32),
        pltpu.VMEM((CB_ * NEG_,), jnp.int32),
        pltpu.VMEM((G_, PW_), jnp.float32),
        pltpu.VMEM((G_, PW_), jnp.float32),
        pltpu.VMEM((PAIRS_, PW_), jnp.float32),
        pltpu.VMEM((CB_,), jnp.float32),
        pltpu.VMEM((CB_ * NEG_,), jnp.float32),
        pltpu.SemaphoreType.DMA,
    ],
)(_sc_body)


@jax.jit
def kernel(w, c, negs, in_table, out_table):
    w32 = w.astype(jnp.int32)
    c32 = c.astype(jnp.int32)
    negs_flat = negs.astype(jnp.int32).reshape(B_ * NEG_)
    in_rows = _to_rows(in_table)
    out_rows = _to_rows(out_table)
    pos_flat, neg_flat = _sc_call(w32, c32, negs_flat, in_rows, out_rows)
    return (pos_flat.reshape(B_, 1), neg_flat.reshape(B_, NEG_, 1))


# speedup vs baseline: 1.0943x; 1.0943x over previous
import functools

import jax
import jax.numpy as jnp
from jax import lax
from jax.experimental import pallas as pl
from jax.experimental.pallas import tpu as pltpu
from jax.experimental.pallas import tpu_sc as plsc

VOCAB_ = 1000000
B_ = 16384
D_ = 64
PW_ = 128
NEG_ = 20
L_ = 16
NC_ = 2
NS_ = 16
NW_ = NC_ * NS_
CB_ = B_ // NW_
G_ = 16
NCHUNK_ = CB_ // G_
PAIRS_ = G_ * NEG_
SEGS_ = (128, 128, 64)
KD_ = D_ // L_

TRB_ = 3968
NTRB_ = 127
SSPLIT_ = TRB_ * NTRB_


def _tr_body(xa_ref, xb_ref, o_ref):
    eye = jnp.eye(D_, dtype=jnp.float32)
    dims = (((0,), (0,)), ((), ()))
    ya = lax.dot_general(xa_ref[...], eye, dims,
                         preferred_element_type=jnp.float32)
    yb = lax.dot_general(xb_ref[...], eye, dims,
                         preferred_element_type=jnp.float32)
    o_ref[...] = jnp.concatenate([ya, yb], axis=1)


_tr = pl.pallas_call(
    _tr_body,
    grid=(NTRB_,),
    in_specs=[
        pl.BlockSpec((D_, TRB_), lambda i: (0, i)),
        pl.BlockSpec((D_, TRB_),
                     lambda i: (0, jnp.minimum(i + NTRB_, VOCAB_ // TRB_))),
    ],
    out_specs=pl.BlockSpec((TRB_, PW_), lambda i: (i, 0)),
    out_shape=jax.ShapeDtypeStruct((SSPLIT_, PW_), jnp.float32),
)


def _to_rows(table):
    tt = jnp.swapaxes(table, 0, 1)
    return _tr(tt, tt)


def _issue(g, in_hbm, out_hbm, row_w, row_c, row_n, wi, wo, wn, sem):
    cps = [
        pltpu.async_copy(in_hbm.at[row_w.at[pl.ds(g * G_, G_)]], wi, sem),
        pltpu.async_copy(out_hbm.at[row_c.at[pl.ds(g * G_, G_)]], wo, sem),
    ]
    off = 0
    for seg in SEGS_:
        cps.append(pltpu.async_copy(
            out_hbm.at[row_n.at[pl.ds(g * PAIRS_ + off, seg)]],
            wn.at[pl.ds(off, seg)], sem))
        off += seg
    return cps


def _sc_body(w_hbm, c_hbm, negs_hbm, in_hbm, out_hbm,
             pos_hbm, neg_hbm,
             idx_w, idx_c, idx_n, row_w, row_c, row_n,
             wi0, wo0, wn0, wi1, wo1, wn1,
             p_pos, p_neg, pos_buf, neg_buf, sem0, sem1):
    cid = lax.axis_index("c")
    sid = lax.axis_index("s")
    wid = sid * NC_ + cid
    base = wid * CB_
    nbase = wid * (CB_ * NEG_)

    bufs = ((wi0, wo0, wn0, sem0), (wi1, wo1, wn1, sem1))

    pltpu.sync_copy(w_hbm.at[pl.ds(base, CB_)], idx_w.at[pl.ds(0, CB_)])
    pltpu.sync_copy(c_hbm.at[pl.ds(base, CB_)], idx_c.at[pl.ds(0, CB_)])
    pltpu.sync_copy(negs_hbm.at[pl.ds(nbase, CB_ * NEG_)],
                    idx_n.at[pl.ds(0, CB_ * NEG_)])

    def mk_rows(i, c2):
        tw = idx_w[pl.ds(i * L_, L_)]
        row_w[pl.ds(i * L_, L_)] = tw - jnp.where(tw >= SSPLIT_, SSPLIT_, 0)
        tc = idx_c[pl.ds(i * L_, L_)]
        row_c[pl.ds(i * L_, L_)] = tc - jnp.where(tc >= SSPLIT_, SSPLIT_, 0)
        return c2
    lax.fori_loop(0, CB_ // L_, mk_rows, 0)

    def mk_rows_n(i, c2):
        tn = idx_n[pl.ds(i * L_, L_)]
        row_n[pl.ds(i * L_, L_)] = tn - jnp.where(tn >= SSPLIT_, SSPLIT_, 0)
        return c2
    lax.fori_loop(0, (CB_ * NEG_) // L_, mk_rows_n, 0)

    iota = lax.iota(jnp.int32, L_)

    def issue(g, s):
        wi, wo, wn, sem = bufs[s]
        _issue(g, in_hbm, out_hbm, row_w, row_c, row_n, wi, wo, wn, sem)

    def wait(g, s):
        wi, wo, wn, sem = bufs[s]
        pltpu.make_async_copy(
            in_hbm.at[row_w.at[pl.ds(g * G_, G_)]], wi, sem).wait()
        pltpu.make_async_copy(
            out_hbm.at[row_c.at[pl.ds(g * G_, G_)]], wo, sem).wait()
        off = 0
        for seg in SEGS_:
            pltpu.make_async_copy(
                out_hbm.at[row_n.at[pl.ds(g * PAIRS_ + off, seg)]],
                wn.at[pl.ds(off, seg)], sem).wait()
            off += seg

    def compute(g, s):
        wi_v, wo_v, wn_v, _ = bufs[s]

        def elem(j, carry2):
            acc_neg, acc_pos = carry2
            vw = idx_w[pl.ds(g * G_ + j, L_)]
            hw = jnp.where(vw[0] >= SSPLIT_, D_, 0)
            wis = [wi_v[j, pl.ds(hw + k * L_, L_)] for k in range(KD_)]
            vc = idx_c[pl.ds(g * G_ + j, L_)]
            hc = jnp.where(vc[0] >= SSPLIT_, D_, 0)
            pacc = wo_v[j, pl.ds(hc, L_)] * wis[0]
            for k in range(1, KD_):
                pacc = pacc + wo_v[j, pl.ds(hc + k * L_, L_)] * wis[k]
            lane_p = j & (L_ - 1)
            acc_pos = jnp.where(iota == lane_p, jnp.sum(pacc), acc_pos)
            pos_buf[pl.ds(g * G_ + j - lane_p, L_)] = acc_pos
            p0 = g * PAIRS_ + j * NEG_
            vn0 = idx_n[pl.ds(p0, L_)]
            vn1 = idx_n[pl.ds(p0 + 8, L_)]
            for n in range(NEG_):
                p = j * NEG_ + n
                hbit = vn0[n] if n < L_ else vn1[n - 8]
                hn = jnp.where(hbit >= SSPLIT_, D_, 0)
                a = wn_v[p, pl.ds(hn, L_)] * wis[0]
                for k in range(1, KD_):
                    a = a + wn_v[p, pl.ds(hn + k * L_, L_)] * wis[k]
                lane = p & (L_ - 1)
                acc_neg = jnp.where(iota == lane, jnp.sum(a), acc_neg)
                neg_buf[pl.ds(g * PAIRS_ + p - lane, L_)] = acc_neg
            return (acc_neg, acc_pos)
        zero = jnp.zeros((L_,), jnp.float32)
        lax.fori_loop(0, G_, elem, (zero, zero))

    issue(0, 0)

    def outer(gg, c2):
        g = gg * 2
        issue(g + 1, 1)
        wait(g, 0)
        compute(g, 0)
        issue(g + 2, 0)
        wait(g + 1, 1)
        compute(g + 1, 1)
        return c2
    lax.fori_loop(0, NCHUNK_ // 2 - 1, outer, 0)

    g_last = NCHUNK_ - 2
    issue(g_last + 1, 1)
    wait(g_last, 0)
    compute(g_last, 0)
    wait(g_last + 1, 1)
    compute(g_last + 1, 1)

    def sig_pos(i, c2):
        v = pos_buf[pl.ds(i * L_, L_)]
        pos_buf[pl.ds(i * L_, L_)] = 1.0 / (1.0 + jnp.exp(-v))
        return c2
    lax.fori_loop(0, CB_ // L_, sig_pos, 0)

    def sig_neg(i, c2):
        v = neg_buf[pl.ds(i * L_, L_)]
        neg_buf[pl.ds(i * L_, L_)] = 1.0 / (1.0 + jnp.exp(v))
        return c2
    lax.fori_loop(0, (CB_ * NEG_) // L_, sig_neg, 0)

    pltpu.sync_copy(pos_buf, pos_hbm.at[pl.ds(base, CB_)])
    pltpu.sync_copy(neg_buf, neg_hbm.at[pl.ds(nbase, CB_ * NEG_)])


_sc_call = functools.partial(
    pl.kernel,
    out_type=(
        jax.ShapeDtypeStruct((B_,), jnp.float32),
        jax.ShapeDtypeStruct((B_ * NEG_,), jnp.float32),
    ),
    mesh=plsc.VectorSubcoreMesh(core_axis_name="c", subcore_axis_name="s"),
    compiler_params=pltpu.CompilerParams(
        needs_layout_passes=False, use_tc_tiling_on_sc=False),
    scratch_types=[
        pltpu.VMEM((CB_ + L_,), jnp.int32),
        pltpu.VMEM((CB_ + L_,), jnp.int32),
        pltpu.VMEM((CB_ * NEG_ + L_,), jnp.int32),
        pltpu.VMEM((CB_,), jnp.int32),
        pltpu.VMEM((CB_,), jnp.int32),
        pltpu.VMEM((CB_ * NEG_,), jnp.int32),
        pltpu.VMEM((G_, PW_), jnp.float32),
        pltpu.VMEM((G_, PW_), jnp.float32),
        pltpu.VMEM((PAIRS_, PW_), jnp.float32),
        pltpu.VMEM((G_, PW_), jnp.float32),
        pltpu.VMEM((G_, PW_), jnp.float32),
        pltpu.VMEM((PAIRS_, PW_), jnp.float32),
        pltpu.VMEM((G_ * L_,), jnp.float32),
        pltpu.VMEM((PAIRS_ * L_,), jnp.float32),
        pltpu.VMEM((CB_,), jnp.float32),
        pltpu.VMEM((CB_ * NEG_,), jnp.float32),
        pltpu.SemaphoreType.DMA,
        pltpu.SemaphoreType.DMA,
    ],
)(_sc_body)


@jax.jit
def kernel(w, c, negs, in_table, out_table):
    w32 = w.astype(jnp.int32)
    c32 = c.astype(jnp.int32)
    negs_flat = negs.astype(jnp.int32).reshape(B_ * NEG_)
    in_rows = _to_rows(in_table)
    out_rows = _to_rows(out_table)
    pos_flat, neg_flat = _sc_call(w32, c32, negs_flat, in_rows, out_rows)
    return (pos_flat.reshape(B_, 1), neg_flat.reshape(B_, NEG_, 1))


# speedup vs baseline: 1.1940x; 1.0911x over previous
import functools

import jax
import jax.numpy as jnp
from jax import lax
from jax.experimental import pallas as pl
from jax.experimental.pallas import tpu as pltpu
from jax.experimental.pallas import tpu_sc as plsc

VOCAB_ = 1000000
B_ = 16384
D_ = 64
PW_ = 128
NEG_ = 20
L_ = 16
NC_ = 2
NS_ = 16
NW_ = NC_ * NS_
CB_ = B_ // NW_
G_ = 16
NCHUNK_ = CB_ // G_
PAIRS_ = G_ * NEG_
SEGS_ = (128, 128, 64)
KD_ = D_ // L_

TRB_ = 7936
NTRB_ = 64
SSPLIT_ = TRB_ * NTRB_


def _tr_body(xa_ref, xb_ref, o_ref):
    eye = jnp.eye(D_, dtype=jnp.float32)
    dims = (((0,), (0,)), ((), ()))
    ya = lax.dot_general(xa_ref[...], eye, dims,
                         preferred_element_type=jnp.float32)
    yb = lax.dot_general(xb_ref[...], eye, dims,
                         preferred_element_type=jnp.float32)
    o_ref[...] = jnp.concatenate([ya, yb], axis=1)


_tr = pl.pallas_call(
    _tr_body,
    grid=(NTRB_,),
    in_specs=[
        pl.BlockSpec((D_, TRB_), lambda i: (0, i)),
        pl.BlockSpec((D_, TRB_),
                     lambda i: (0, jnp.minimum(i + NTRB_, VOCAB_ // TRB_))),
    ],
    out_specs=pl.BlockSpec((TRB_, PW_), lambda i: (i, 0)),
    out_shape=jax.ShapeDtypeStruct((SSPLIT_, PW_), jnp.float32),
)


def _to_rows(table):
    tt = jnp.swapaxes(table, 0, 1)
    return _tr(tt, tt)


def _issue(g, in_hbm, out_hbm, row_w, row_c, row_n, wi, wo, wn, sem):
    cps = [
        pltpu.async_copy(in_hbm.at[row_w.at[pl.ds(g * G_, G_)]], wi, sem),
        pltpu.async_copy(out_hbm.at[row_c.at[pl.ds(g * G_, G_)]], wo, sem),
    ]
    off = 0
    for seg in SEGS_:
        cps.append(pltpu.async_copy(
            out_hbm.at[row_n.at[pl.ds(g * PAIRS_ + off, seg)]],
            wn.at[pl.ds(off, seg)], sem))
        off += seg
    return cps


def _sc_body(w_hbm, c_hbm, negs_hbm, in_hbm, out_hbm,
             pos_hbm, neg_hbm,
             idx_w, idx_c, idx_n, row_w, row_c, row_n,
             wi0, wo0, wn0, wi1, wo1, wn1,
             p_pos, p_neg, pos_buf, neg_buf, sem0, sem1):
    cid = lax.axis_index("c")
    sid = lax.axis_index("s")
    wid = sid * NC_ + cid
    base = wid * CB_
    nbase = wid * (CB_ * NEG_)

    bufs = ((wi0, wo0, wn0, sem0), (wi1, wo1, wn1, sem1))

    pltpu.sync_copy(w_hbm.at[pl.ds(base, CB_)], idx_w.at[pl.ds(0, CB_)])
    pltpu.sync_copy(c_hbm.at[pl.ds(base, CB_)], idx_c.at[pl.ds(0, CB_)])
    pltpu.sync_copy(negs_hbm.at[pl.ds(nbase, CB_ * NEG_)],
                    idx_n.at[pl.ds(0, CB_ * NEG_)])

    def mk_rows(i, c2):
        tw = idx_w[pl.ds(i * L_, L_)]
        row_w[pl.ds(i * L_, L_)] = tw - jnp.where(tw >= SSPLIT_, SSPLIT_, 0)
        tc = idx_c[pl.ds(i * L_, L_)]
        row_c[pl.ds(i * L_, L_)] = tc - jnp.where(tc >= SSPLIT_, SSPLIT_, 0)
        return c2
    lax.fori_loop(0, CB_ // L_, mk_rows, 0)

    def mk_rows_n(i, c2):
        tn = idx_n[pl.ds(i * L_, L_)]
        row_n[pl.ds(i * L_, L_)] = tn - jnp.where(tn >= SSPLIT_, SSPLIT_, 0)
        return c2
    lax.fori_loop(0, (CB_ * NEG_) // L_, mk_rows_n, 0)

    iota = lax.iota(jnp.int32, L_)

    def issue(g, s):
        wi, wo, wn, sem = bufs[s]
        _issue(g, in_hbm, out_hbm, row_w, row_c, row_n, wi, wo, wn, sem)

    def wait(g, s):
        wi, wo, wn, sem = bufs[s]
        pltpu.make_async_copy(
            in_hbm.at[row_w.at[pl.ds(g * G_, G_)]], wi, sem).wait()
        pltpu.make_async_copy(
            out_hbm.at[row_c.at[pl.ds(g * G_, G_)]], wo, sem).wait()
        off = 0
        for seg in SEGS_:
            pltpu.make_async_copy(
                out_hbm.at[row_n.at[pl.ds(g * PAIRS_ + off, seg)]],
                wn.at[pl.ds(off, seg)], sem).wait()
            off += seg

    def compute(g, s):
        wi_v, wo_v, wn_v, _ = bufs[s]

        def elem(j, carry2):
            acc_neg, acc_pos = carry2
            vw = idx_w[pl.ds(g * G_ + j, L_)]
            hw = jnp.where(vw[0] >= SSPLIT_, D_, 0)
            wis = [wi_v[j, pl.ds(hw + k * L_, L_)] for k in range(KD_)]
            vc = idx_c[pl.ds(g * G_ + j, L_)]
            hc = jnp.where(vc[0] >= SSPLIT_, D_, 0)
            pacc = wo_v[j, pl.ds(hc, L_)] * wis[0]
            for k in range(1, KD_):
                pacc = pacc + wo_v[j, pl.ds(hc + k * L_, L_)] * wis[k]
            lane_p = j & (L_ - 1)
            acc_pos = jnp.where(iota == lane_p, jnp.sum(pacc), acc_pos)
            pos_buf[pl.ds(g * G_ + j - lane_p, L_)] = acc_pos
            p0 = g * PAIRS_ + j * NEG_
            vn0 = idx_n[pl.ds(p0, L_)]
            vn1 = idx_n[pl.ds(p0 + 8, L_)]
            for n in range(NEG_):
                p = j * NEG_ + n
                hbit = vn0[n] if n < L_ else vn1[n - 8]
                hn = jnp.where(hbit >= SSPLIT_, D_, 0)
                a = wn_v[p, pl.ds(hn, L_)] * wis[0]
                for k in range(1, KD_):
                    a = a + wn_v[p, pl.ds(hn + k * L_, L_)] * wis[k]
                lane = p & (L_ - 1)
                acc_neg = jnp.where(iota == lane, jnp.sum(a), acc_neg)
                neg_buf[pl.ds(g * PAIRS_ + p - lane, L_)] = acc_neg
            return (acc_neg, acc_pos)
        zero = jnp.zeros((L_,), jnp.float32)
        lax.fori_loop(0, G_, elem, (zero, zero))

    issue(0, 0)

    def outer(gg, c2):
        g = gg * 2
        issue(g + 1, 1)
        wait(g, 0)
        compute(g, 0)
        issue(g + 2, 0)
        wait(g + 1, 1)
        compute(g + 1, 1)
        return c2
    lax.fori_loop(0, NCHUNK_ // 2 - 1, outer, 0)

    g_last = NCHUNK_ - 2
    issue(g_last + 1, 1)
    wait(g_last, 0)
    compute(g_last, 0)
    wait(g_last + 1, 1)
    compute(g_last + 1, 1)

    def sig_pos(i, c2):
        v = pos_buf[pl.ds(i * L_, L_)]
        pos_buf[pl.ds(i * L_, L_)] = 1.0 / (1.0 + jnp.exp(-v))
        return c2
    lax.fori_loop(0, CB_ // L_, sig_pos, 0)

    def sig_neg(i, c2):
        v = neg_buf[pl.ds(i * L_, L_)]
        neg_buf[pl.ds(i * L_, L_)] = 1.0 / (1.0 + jnp.exp(v))
        return c2
    lax.fori_loop(0, (CB_ * NEG_) // L_, sig_neg, 0)

    pltpu.sync_copy(pos_buf, pos_hbm.at[pl.ds(base, CB_)])
    pltpu.sync_copy(neg_buf, neg_hbm.at[pl.ds(nbase, CB_ * NEG_)])


_sc_call = functools.partial(
    pl.kernel,
    out_type=(
        jax.ShapeDtypeStruct((B_,), jnp.float32),
        jax.ShapeDtypeStruct((B_ * NEG_,), jnp.float32),
    ),
    mesh=plsc.VectorSubcoreMesh(core_axis_name="c", subcore_axis_name="s"),
    compiler_params=pltpu.CompilerParams(
        needs_layout_passes=False, use_tc_tiling_on_sc=False),
    scratch_types=[
        pltpu.VMEM((CB_ + L_,), jnp.int32),
        pltpu.VMEM((CB_ + L_,), jnp.int32),
        pltpu.VMEM((CB_ * NEG_ + L_,), jnp.int32),
        pltpu.VMEM((CB_,), jnp.int32),
        pltpu.VMEM((CB_,), jnp.int32),
        pltpu.VMEM((CB_ * NEG_,), jnp.int32),
        pltpu.VMEM((G_, PW_), jnp.float32),
        pltpu.VMEM((G_, PW_), jnp.float32),
        pltpu.VMEM((PAIRS_, PW_), jnp.float32),
        pltpu.VMEM((G_, PW_), jnp.float32),
        pltpu.VMEM((G_, PW_), jnp.float32),
        pltpu.VMEM((PAIRS_, PW_), jnp.float32),
        pltpu.VMEM((G_ * L_,), jnp.float32),
        pltpu.VMEM((PAIRS_ * L_,), jnp.float32),
        pltpu.VMEM((CB_,), jnp.float32),
        pltpu.VMEM((CB_ * NEG_,), jnp.float32),
        pltpu.SemaphoreType.DMA,
        pltpu.SemaphoreType.DMA,
    ],
)(_sc_body)


@jax.jit
def kernel(w, c, negs, in_table, out_table):
    w32 = w.astype(jnp.int32)
    c32 = c.astype(jnp.int32)
    negs_flat = negs.astype(jnp.int32).reshape(B_ * NEG_)
    in_rows = _to_rows(in_table)
    out_rows = _to_rows(out_table)
    pos_flat, neg_flat = _sc_call(w32, c32, negs_flat, in_rows, out_rows)
    return (pos_flat.reshape(B_, 1), neg_flat.reshape(B_, NEG_, 1))


# speedup vs baseline: 1.1950x; 1.0009x over previous
import functools

import jax
import jax.numpy as jnp
from jax import lax
from jax.experimental import pallas as pl
from jax.experimental.pallas import tpu as pltpu
from jax.experimental.pallas import tpu_sc as plsc

VOCAB_ = 1000000
B_ = 16384
D_ = 64
PW_ = 128
NEG_ = 20
L_ = 16
NC_ = 2
NS_ = 16
NW_ = NC_ * NS_
CB_ = B_ // NW_
G_ = 16
NCHUNK_ = CB_ // G_
PAIRS_ = G_ * NEG_
SEGS_ = (128, 128, 64)
KD_ = D_ // L_

TRB_ = 7936
NTRB_ = 64
SSPLIT_ = TRB_ * NTRB_


def _tr_body(xa_ref, xb_ref, o_ref):
    ya = lax.transpose(xa_ref[...], (1, 0))
    yb = lax.transpose(xb_ref[...], (1, 0))
    o_ref[...] = jnp.concatenate([ya, yb], axis=1)


_tr = pl.pallas_call(
    _tr_body,
    grid=(NTRB_,),
    in_specs=[
        pl.BlockSpec((D_, TRB_), lambda i: (0, i)),
        pl.BlockSpec((D_, TRB_),
                     lambda i: (0, jnp.minimum(i + NTRB_, VOCAB_ // TRB_))),
    ],
    out_specs=pl.BlockSpec((TRB_, PW_), lambda i: (i, 0)),
    out_shape=jax.ShapeDtypeStruct((SSPLIT_, PW_), jnp.float32),
)


def _to_rows(table):
    tt = jnp.swapaxes(table, 0, 1)
    return _tr(tt, tt)


def _issue(g, in_hbm, out_hbm, row_w, row_c, row_n, wi, wo, wn, sem):
    cps = [
        pltpu.async_copy(in_hbm.at[row_w.at[pl.ds(g * G_, G_)]], wi, sem),
        pltpu.async_copy(out_hbm.at[row_c.at[pl.ds(g * G_, G_)]], wo, sem),
    ]
    off = 0
    for seg in SEGS_:
        cps.append(pltpu.async_copy(
            out_hbm.at[row_n.at[pl.ds(g * PAIRS_ + off, seg)]],
            wn.at[pl.ds(off, seg)], sem))
        off += seg
    return cps


def _sc_body(w_hbm, c_hbm, negs_hbm, in_hbm, out_hbm,
             pos_hbm, neg_hbm,
             idx_w, idx_c, idx_n, row_w, row_c, row_n,
             wi0, wo0, wn0, wi1, wo1, wn1,
             p_pos, p_neg, pos_buf, neg_buf, sem0, sem1):
    cid = lax.axis_index("c")
    sid = lax.axis_index("s")
    wid = sid * NC_ + cid
    base = wid * CB_
    nbase = wid * (CB_ * NEG_)

    bufs = ((wi0, wo0, wn0, sem0), (wi1, wo1, wn1, sem1))

    pltpu.sync_copy(w_hbm.at[pl.ds(base, CB_)], idx_w.at[pl.ds(0, CB_)])
    pltpu.sync_copy(c_hbm.at[pl.ds(base, CB_)], idx_c.at[pl.ds(0, CB_)])
    pltpu.sync_copy(negs_hbm.at[pl.ds(nbase, CB_ * NEG_)],
                    idx_n.at[pl.ds(0, CB_ * NEG_)])

    def mk_rows(i, c2):
        tw = idx_w[pl.ds(i * L_, L_)]
        row_w[pl.ds(i * L_, L_)] = tw - jnp.where(tw >= SSPLIT_, SSPLIT_, 0)
        tc = idx_c[pl.ds(i * L_, L_)]
        row_c[pl.ds(i * L_, L_)] = tc - jnp.where(tc >= SSPLIT_, SSPLIT_, 0)
        return c2
    lax.fori_loop(0, CB_ // L_, mk_rows, 0)

    def mk_rows_n(i, c2):
        tn = idx_n[pl.ds(i * L_, L_)]
        row_n[pl.ds(i * L_, L_)] = tn - jnp.where(tn >= SSPLIT_, SSPLIT_, 0)
        return c2
    lax.fori_loop(0, (CB_ * NEG_) // L_, mk_rows_n, 0)

    iota = lax.iota(jnp.int32, L_)

    def issue(g, s):
        wi, wo, wn, sem = bufs[s]
        _issue(g, in_hbm, out_hbm, row_w, row_c, row_n, wi, wo, wn, sem)

    def wait(g, s):
        wi, wo, wn, sem = bufs[s]
        pltpu.make_async_copy(
            in_hbm.at[row_w.at[pl.ds(g * G_, G_)]], wi, sem).wait()
        pltpu.make_async_copy(
            out_hbm.at[row_c.at[pl.ds(g * G_, G_)]], wo, sem).wait()
        off = 0
        for seg in SEGS_:
            pltpu.make_async_copy(
                out_hbm.at[row_n.at[pl.ds(g * PAIRS_ + off, seg)]],
                wn.at[pl.ds(off, seg)], sem).wait()
            off += seg

    def compute(g, s):
        wi_v, wo_v, wn_v, _ = bufs[s]

        def elem(j, carry2):
            acc_neg, acc_pos = carry2
            vw = idx_w[pl.ds(g * G_ + j, L_)]
            hw = jnp.where(vw[0] >= SSPLIT_, D_, 0)
            wis = [wi_v[j, pl.ds(hw + k * L_, L_)] for k in range(KD_)]
            vc = idx_c[pl.ds(g * G_ + j, L_)]
            hc = jnp.where(vc[0] >= SSPLIT_, D_, 0)
            pacc = wo_v[j, pl.ds(hc, L_)] * wis[0]
            for k in range(1, KD_):
                pacc = pacc + wo_v[j, pl.ds(hc + k * L_, L_)] * wis[k]
            lane_p = j & (L_ - 1)
            acc_pos = jnp.where(iota == lane_p, jnp.sum(pacc), acc_pos)
            pos_buf[pl.ds(g * G_ + j - lane_p, L_)] = acc_pos
            p0 = g * PAIRS_ + j * NEG_
            vn0 = idx_n[pl.ds(p0, L_)]
            vn1 = idx_n[pl.ds(p0 + 8, L_)]
            for n in range(NEG_):
                p = j * NEG_ + n
                hbit = vn0[n] if n < L_ else vn1[n - 8]
                hn = jnp.where(hbit >= SSPLIT_, D_, 0)
                a = wn_v[p, pl.ds(hn, L_)] * wis[0]
                for k in range(1, KD_):
                    a = a + wn_v[p, pl.ds(hn + k * L_, L_)] * wis[k]
                lane = p & (L_ - 1)
                acc_neg = jnp.where(iota == lane, jnp.sum(a), acc_neg)
                neg_buf[pl.ds(g * PAIRS_ + p - lane, L_)] = acc_neg
            return (acc_neg, acc_pos)
        zero = jnp.zeros((L_,), jnp.float32)
        lax.fori_loop(0, G_, elem, (zero, zero))

    issue(0, 0)

    def outer(gg, c2):
        g = gg * 2
        issue(g + 1, 1)
        wait(g, 0)
        compute(g, 0)
        issue(g + 2, 0)
        wait(g + 1, 1)
        compute(g + 1, 1)
        return c2
    lax.fori_loop(0, NCHUNK_ // 2 - 1, outer, 0)

    g_last = NCHUNK_ - 2
    issue(g_last + 1, 1)
    wait(g_last, 0)
    compute(g_last, 0)
    wait(g_last + 1, 1)
    compute(g_last + 1, 1)

    def sig_pos(i, c2):
        v = pos_buf[pl.ds(i * L_, L_)]
        pos_buf[pl.ds(i * L_, L_)] = 1.0 / (1.0 + jnp.exp(-v))
        return c2
    lax.fori_loop(0, CB_ // L_, sig_pos, 0)

    def sig_neg(i, c2):
        v = neg_buf[pl.ds(i * L_, L_)]
        neg_buf[pl.ds(i * L_, L_)] = 1.0 / (1.0 + jnp.exp(v))
        return c2
    lax.fori_loop(0, (CB_ * NEG_) // L_, sig_neg, 0)

    pltpu.sync_copy(pos_buf, pos_hbm.at[pl.ds(base, CB_)])
    pltpu.sync_copy(neg_buf, neg_hbm.at[pl.ds(nbase, CB_ * NEG_)])


_sc_call = functools.partial(
    pl.kernel,
    out_type=(
        jax.ShapeDtypeStruct((B_,), jnp.float32),
        jax.ShapeDtypeStruct((B_ * NEG_,), jnp.float32),
    ),
    mesh=plsc.VectorSubcoreMesh(core_axis_name="c", subcore_axis_name="s"),
    compiler_params=pltpu.CompilerParams(
        needs_layout_passes=False, use_tc_tiling_on_sc=False),
    scratch_types=[
        pltpu.VMEM((CB_ + L_,), jnp.int32),
        pltpu.VMEM((CB_ + L_,), jnp.int32),
        pltpu.VMEM((CB_ * NEG_ + L_,), jnp.int32),
        pltpu.VMEM((CB_,), jnp.int32),
        pltpu.VMEM((CB_,), jnp.int32),
        pltpu.VMEM((CB_ * NEG_,), jnp.int32),
        pltpu.VMEM((G_, PW_), jnp.float32),
        pltpu.VMEM((G_, PW_), jnp.float32),
        pltpu.VMEM((PAIRS_, PW_), jnp.float32),
        pltpu.VMEM((G_, PW_), jnp.float32),
        pltpu.VMEM((G_, PW_), jnp.float32),
        pltpu.VMEM((PAIRS_, PW_), jnp.float32),
        pltpu.VMEM((G_ * L_,), jnp.float32),
        pltpu.VMEM((PAIRS_ * L_,), jnp.float32),
        pltpu.VMEM((CB_,), jnp.float32),
        pltpu.VMEM((CB_ * NEG_,), jnp.float32),
        pltpu.SemaphoreType.DMA,
        pltpu.SemaphoreType.DMA,
    ],
)(_sc_body)


@jax.jit
def kernel(w, c, negs, in_table, out_table):
    w32 = w.astype(jnp.int32)
    c32 = c.astype(jnp.int32)
    negs_flat = negs.astype(jnp.int32).reshape(B_ * NEG_)
    in_rows = _to_rows(in_table)
    out_rows = _to_rows(out_table)
    pos_flat, neg_flat = _sc_call(w32, c32, negs_flat, in_rows, out_rows)
    return (pos_flat.reshape(B_, 1), neg_flat.reshape(B_, NEG_, 1))


# speedup vs baseline: 1.2238x; 1.0241x over previous
import functools

import jax
import jax.numpy as jnp
from jax import lax
from jax.experimental import pallas as pl
from jax.experimental.pallas import tpu as pltpu
from jax.experimental.pallas import tpu_sc as plsc

VOCAB_ = 1000000
B_ = 16384
D_ = 64
PW_ = 128
NEG_ = 20
L_ = 16
NC_ = 2
NS_ = 16
NW_ = NC_ * NS_
CB_ = B_ // NW_
G_ = 16
NCHUNK_ = CB_ // G_
PAIRS_ = G_ * NEG_
SEGS_ = (128, 128, 64)
KD_ = D_ // L_

TRB_ = 7936
NTRB_ = 64
SSPLIT_ = TRB_ * NTRB_


def _tr_body(xa_ref, xb_ref, o_ref):
    ya = lax.transpose(xa_ref[...], (1, 0))
    yb = lax.transpose(xb_ref[...], (1, 0))
    o_ref[...] = jnp.concatenate([ya, yb], axis=1)


_tr = pl.pallas_call(
    _tr_body,
    grid=(NTRB_,),
    in_specs=[
        pl.BlockSpec((D_, TRB_), lambda i: (0, i)),
        pl.BlockSpec((D_, TRB_),
                     lambda i: (0, jnp.minimum(i + NTRB_, VOCAB_ // TRB_))),
    ],
    out_specs=pl.BlockSpec((TRB_, PW_), lambda i: (i, 0)),
    out_shape=jax.ShapeDtypeStruct((SSPLIT_, PW_), jnp.float32),
)


def _to_rows(table):
    tt = jnp.swapaxes(table, 0, 1)
    return _tr(tt, tt)


def _issue(g, in_hbm, out_hbm, row_w, row_c, row_n, wi, wo, wn, sem):
    cps = [
        pltpu.async_copy(in_hbm.at[row_w.at[pl.ds(g * G_, G_)]], wi, sem),
        pltpu.async_copy(out_hbm.at[row_c.at[pl.ds(g * G_, G_)]], wo, sem),
    ]
    off = 0
    for seg in SEGS_:
        cps.append(pltpu.async_copy(
            out_hbm.at[row_n.at[pl.ds(g * PAIRS_ + off, seg)]],
            wn.at[pl.ds(off, seg)], sem))
        off += seg
    return cps


def _sc_body(w_hbm, c_hbm, negs_hbm, in_hbm, out_hbm,
             pos_hbm, neg_hbm,
             idx_w, idx_c, idx_n, row_w, row_c, row_n,
             wi0, wo0, wn0, wi1, wo1, wn1,
             p_pos, p_neg, pos_buf, neg_buf, sem0, sem1):
    cid = lax.axis_index("c")
    sid = lax.axis_index("s")
    wid = sid * NC_ + cid
    base = wid * CB_
    nbase = wid * (CB_ * NEG_)

    bufs = ((wi0, wo0, wn0, sem0), (wi1, wo1, wn1, sem1))

    pltpu.sync_copy(w_hbm.at[pl.ds(base, CB_)], idx_w.at[pl.ds(0, CB_)])
    pltpu.sync_copy(c_hbm.at[pl.ds(base, CB_)], idx_c.at[pl.ds(0, CB_)])
    pltpu.sync_copy(negs_hbm.at[pl.ds(nbase, CB_ * NEG_)],
                    idx_n.at[pl.ds(0, CB_ * NEG_)])

    def mk_rows(i, c2):
        tw = idx_w[pl.ds(i * L_, L_)]
        row_w[pl.ds(i * L_, L_)] = (
            2 * tw - jnp.where(tw >= SSPLIT_, 2 * SSPLIT_ - 1, 0))
        tc = idx_c[pl.ds(i * L_, L_)]
        row_c[pl.ds(i * L_, L_)] = (
            2 * tc - jnp.where(tc >= SSPLIT_, 2 * SSPLIT_ - 1, 0))
        return c2
    lax.fori_loop(0, CB_ // L_, mk_rows, 0)

    def mk_rows_n(i, c2):
        tn = idx_n[pl.ds(i * L_, L_)]
        row_n[pl.ds(i * L_, L_)] = (
            2 * tn - jnp.where(tn >= SSPLIT_, 2 * SSPLIT_ - 1, 0))
        return c2
    lax.fori_loop(0, (CB_ * NEG_) // L_, mk_rows_n, 0)

    iota = lax.iota(jnp.int32, L_)

    def issue(g, s):
        wi, wo, wn, sem = bufs[s]
        _issue(g, in_hbm, out_hbm, row_w, row_c, row_n, wi, wo, wn, sem)

    def wait(g, s):
        wi, wo, wn, sem = bufs[s]
        pltpu.make_async_copy(
            in_hbm.at[row_w.at[pl.ds(g * G_, G_)]], wi, sem).wait()
        pltpu.make_async_copy(
            out_hbm.at[row_c.at[pl.ds(g * G_, G_)]], wo, sem).wait()
        off = 0
        for seg in SEGS_:
            pltpu.make_async_copy(
                out_hbm.at[row_n.at[pl.ds(g * PAIRS_ + off, seg)]],
                wn.at[pl.ds(off, seg)], sem).wait()
            off += seg

    def compute(g, s):
        wi_v, wo_v, wn_v, _ = bufs[s]

        def elem(j, carry2):
            acc_neg, acc_pos = carry2
            wis = [wi_v[j, pl.ds(k * L_, L_)] for k in range(KD_)]
            pacc = wo_v[j, pl.ds(0, L_)] * wis[0]
            for k in range(1, KD_):
                pacc = pacc + wo_v[j, pl.ds(k * L_, L_)] * wis[k]
            lane_p = j & (L_ - 1)
            acc_pos = jnp.where(iota == lane_p, jnp.sum(pacc), acc_pos)
            pos_buf[pl.ds(g * G_ + j - lane_p, L_)] = acc_pos
            for n in range(NEG_):
                p = j * NEG_ + n
                a = wn_v[p, pl.ds(0, L_)] * wis[0]
                for k in range(1, KD_):
                    a = a + wn_v[p, pl.ds(k * L_, L_)] * wis[k]
                lane = p & (L_ - 1)
                acc_neg = jnp.where(iota == lane, jnp.sum(a), acc_neg)
                neg_buf[pl.ds(g * PAIRS_ + p - lane, L_)] = acc_neg
            return (acc_neg, acc_pos)
        zero = jnp.zeros((L_,), jnp.float32)
        lax.fori_loop(0, G_, elem, (zero, zero))

    issue(0, 0)

    def outer(gg, c2):
        g = gg * 2
        issue(g + 1, 1)
        wait(g, 0)
        compute(g, 0)
        issue(g + 2, 0)
        wait(g + 1, 1)
        compute(g + 1, 1)
        return c2
    lax.fori_loop(0, NCHUNK_ // 2 - 1, outer, 0)

    g_last = NCHUNK_ - 2
    issue(g_last + 1, 1)
    wait(g_last, 0)
    compute(g_last, 0)
    wait(g_last + 1, 1)
    compute(g_last + 1, 1)

    def sig_pos(i, c2):
        v = pos_buf[pl.ds(i * L_, L_)]
        pos_buf[pl.ds(i * L_, L_)] = 1.0 / (1.0 + jnp.exp(-v))
        return c2
    lax.fori_loop(0, CB_ // L_, sig_pos, 0)

    def sig_neg(i, c2):
        v = neg_buf[pl.ds(i * L_, L_)]
        neg_buf[pl.ds(i * L_, L_)] = 1.0 / (1.0 + jnp.exp(v))
        return c2
    lax.fori_loop(0, (CB_ * NEG_) // L_, sig_neg, 0)

    pltpu.sync_copy(pos_buf, pos_hbm.at[pl.ds(base, CB_)])
    pltpu.sync_copy(neg_buf, neg_hbm.at[pl.ds(nbase, CB_ * NEG_)])


_sc_call = functools.partial(
    pl.kernel,
    out_type=(
        jax.ShapeDtypeStruct((B_,), jnp.float32),
        jax.ShapeDtypeStruct((B_ * NEG_,), jnp.float32),
    ),
    mesh=plsc.VectorSubcoreMesh(core_axis_name="c", subcore_axis_name="s"),
    compiler_params=pltpu.CompilerParams(
        needs_layout_passes=False, use_tc_tiling_on_sc=False),
    scratch_types=[
        pltpu.VMEM((CB_ + L_,), jnp.int32),
        pltpu.VMEM((CB_ + L_,), jnp.int32),
        pltpu.VMEM((CB_ * NEG_ + L_,), jnp.int32),
        pltpu.VMEM((CB_,), jnp.int32),
        pltpu.VMEM((CB_,), jnp.int32),
        pltpu.VMEM((CB_ * NEG_,), jnp.int32),
        pltpu.VMEM((G_, D_), jnp.float32),
        pltpu.VMEM((G_, D_), jnp.float32),
        pltpu.VMEM((PAIRS_, D_), jnp.float32),
        pltpu.VMEM((G_, D_), jnp.float32),
        pltpu.VMEM((G_, D_), jnp.float32),
        pltpu.VMEM((PAIRS_, D_), jnp.float32),
        pltpu.VMEM((G_ * L_,), jnp.float32),
        pltpu.VMEM((PAIRS_ * L_,), jnp.float32),
        pltpu.VMEM((CB_,), jnp.float32),
        pltpu.VMEM((CB_ * NEG_,), jnp.float32),
        pltpu.SemaphoreType.DMA,
        pltpu.SemaphoreType.DMA,
    ],
)(_sc_body)


@jax.jit
def kernel(w, c, negs, in_table, out_table):
    w32 = w.astype(jnp.int32)
    c32 = c.astype(jnp.int32)
    negs_flat = negs.astype(jnp.int32).reshape(B_ * NEG_)
    in_rows = _to_rows(in_table).reshape(2 * SSPLIT_, D_)
    out_rows = _to_rows(out_table).reshape(2 * SSPLIT_, D_)
    pos_flat, neg_flat = _sc_call(w32, c32, negs_flat, in_rows, out_rows)
    return (pos_flat.reshape(B_, 1), neg_flat.reshape(B_, NEG_, 1))
